# baseline (speedup 1.0000x reference)
import functools

import jax
import jax.numpy as jnp
from jax import lax
from jax.experimental import pallas as pl
from jax.experimental.pallas import tpu as pltpu

N_NODES = 10000
N_EDGES = 320000
FEAT = 128
BOND = 147
HID = 256

BE = 3200
BN = 2000


def _edge_mm_body(e_ref, w_ref, b_ref, o_ref):
    o_ref[...] = (
        jnp.dot(e_ref[...], w_ref[...], preferred_element_type=jnp.float32, precision=lax.Precision.HIGHEST)
        + b_ref[...]
    )


def edge_mm(e, We, be):
    E, K = e.shape
    return pl.pallas_call(
        _edge_mm_body,
        grid=(E // BE,),
        in_specs=[
            pl.BlockSpec((BE, K), lambda i: (i, 0)),
            pl.BlockSpec((K, HID), lambda i: (0, 0)),
            pl.BlockSpec((1, HID), lambda i: (0, 0)),
        ],
        out_specs=pl.BlockSpec((BE, HID), lambda i: (i, 0)),
        out_shape=jax.ShapeDtypeStruct((E, HID), jnp.float32),
    )(e, We, be.reshape(1, HID))


def _edge_mm_fused_body(hg_ref, p_ref, w_ref, b_ref, o_ref):
    t = jnp.maximum(hg_ref[...] + p_ref[...], 0.0)
    o_ref[...] = (
        jnp.dot(t, w_ref[...], preferred_element_type=jnp.float32, precision=lax.Precision.HIGHEST) + b_ref[...]
    )


def edge_mm_fused(h_src, P_prev, We, be):
    E = h_src.shape[0]
    return pl.pallas_call(
        _edge_mm_fused_body,
        grid=(E // BE,),
        in_specs=[
            pl.BlockSpec((BE, HID), lambda i: (i, 0)),
            pl.BlockSpec((BE, HID), lambda i: (i, 0)),
            pl.BlockSpec((HID, HID), lambda i: (0, 0)),
            pl.BlockSpec((1, HID), lambda i: (0, 0)),
        ],
        out_specs=pl.BlockSpec((BE, HID), lambda i: (i, 0)),
        out_shape=jax.ShapeDtypeStruct((E, HID), jnp.float32),
    )(h_src, P_prev, We, be.reshape(1, HID))


def _node_mm_body(h_ref, s_ref, x_ref, wn_ref, bn_ref, wc_ref, bc_ref, o_ref,
                  *, input_relu):
    x = x_ref[...]
    x = jnp.where(jnp.isfinite(x), x, 0.0)
    m = (
        jnp.dot(s_ref[...], wc_ref[0:HID, :], preferred_element_type=jnp.float32, precision=lax.Precision.HIGHEST)
        + jnp.dot(x, wc_ref[HID:2 * HID, :], preferred_element_type=jnp.float32, precision=lax.Precision.HIGHEST)
        + bc_ref[...]
    )
    hin = h_ref[...]
    if input_relu:
        hin = jnp.maximum(hin, 0.0)
    h = (
        jnp.dot(hin, wn_ref[...], preferred_element_type=jnp.float32, precision=lax.Precision.HIGHEST)
        + bn_ref[...]
        + m
    )
    o_ref[...] = h


def node_mm(h, S, X, Wn, bn, Wc, bc, input_relu):
    N, K = h.shape
    return pl.pallas_call(
        functools.partial(_node_mm_body, input_relu=input_relu),
        grid=(N // BN,),
        in_specs=[
            pl.BlockSpec((BN, K), lambda i: (i, 0)),
            pl.BlockSpec((BN, HID), lambda i: (i, 0)),
            pl.BlockSpec((BN, HID), lambda i: (i, 0)),
            pl.BlockSpec((K, HID), lambda i: (0, 0)),
            pl.BlockSpec((1, HID), lambda i: (0, 0)),
            pl.BlockSpec((2 * HID, HID), lambda i: (0, 0)),
            pl.BlockSpec((1, HID), lambda i: (0, 0)),
        ],
        out_specs=pl.BlockSpec((BN, HID), lambda i: (i, 0)),
        out_shape=jax.ShapeDtypeStruct((N, HID), jnp.float32),
    )(h, S, X, Wn, bn.reshape(1, HID), Wc, bc.reshape(1, HID))


def _epilogue_body(h_ref, o_ref, acc_ref, nrm_ref):
    i = pl.program_id(0)

    @pl.when(i == 0)
    def _init():
        acc_ref[...] = jnp.zeros_like(acc_ref)
        nrm_ref[...] = jnp.zeros_like(nrm_ref)

    h = h_ref[...]
    acc_ref[...] += jnp.sum(h, axis=0, keepdims=True)
    rown = jnp.sqrt(jnp.sum(h * h, axis=1, keepdims=True))
    nrm_ref[...] += jnp.sum(rown, axis=0, keepdims=True)

    @pl.when(i == pl.num_programs(0) - 1)
    def _fin():
        factor = jnp.sqrt(jnp.float32(HID)) * N_NODES / nrm_ref[0, 0]
        o_ref[...] = acc_ref[...] * factor


def epilogue(h):
    return pl.pallas_call(
        _epilogue_body,
        grid=(N_NODES // BN,),
        in_specs=[pl.BlockSpec((BN, HID), lambda i: (i, 0))],
        out_specs=pl.BlockSpec((1, HID), lambda i: (0, 0)),
        out_shape=jax.ShapeDtypeStruct((1, HID), jnp.float32),
        scratch_shapes=[
            pltpu.VMEM((1, HID), jnp.float32),
            pltpu.VMEM((1, 1), jnp.float32),
        ],
    )(h)


def kernel(x, edge_index, edge_attr,
           Wn0, bn0, We0, be0, Wc0, bc0,
           Wn1, bn1, We1, be1, Wc1, bc1,
           Wn2, bn2, We2, be2, Wc2, bc2):
    src = edge_index[0]
    dst = edge_index[1]

    P = edge_mm(edge_attr, We0, be0)
    params = [(Wn0, bn0, We1, be1, Wc0, bc0),
              (Wn1, bn1, We2, be2, Wc1, bc1),
              (Wn2, bn2, None, None, Wc2, bc2)]
    h = x
    for i, (Wn, bn, We_next, be_next, Wc, bc) in enumerate(params):
        S = jax.ops.segment_sum(P, dst, num_segments=N_NODES)
        X = jax.ops.segment_max(P, dst, num_segments=N_NODES)
        h = node_mm(h, S, X, Wn, bn, Wc, bc, input_relu=(i > 0))
        if We_next is not None:
            h_src = h[src]
            P = edge_mm_fused(h_src, P, We_next, be_next)
    return epilogue(h)


# speedup vs baseline: 1.0206x; 1.0206x over previous
import functools

import jax
import jax.numpy as jnp
from jax import lax
from jax.experimental import pallas as pl
from jax.experimental.pallas import tpu as pltpu
from jax.experimental.pallas import tpu_sc as plsc

N_NODES = 10000
N_EDGES = 320000
FEAT = 128
BOND = 147
HID = 256

BE = 3200
BN = 2000


def _edge_mm_body(e_ref, w_ref, b_ref, o_ref):
    o_ref[...] = (
        jnp.dot(e_ref[...], w_ref[...], preferred_element_type=jnp.float32, precision=lax.Precision.HIGHEST)
        + b_ref[...]
    )


def edge_mm(e, We, be):
    E, K = e.shape
    return pl.pallas_call(
        _edge_mm_body,
        grid=(E // BE,),
        in_specs=[
            pl.BlockSpec((BE, K), lambda i: (i, 0)),
            pl.BlockSpec((K, HID), lambda i: (0, 0)),
            pl.BlockSpec((1, HID), lambda i: (0, 0)),
        ],
        out_specs=pl.BlockSpec((BE, HID), lambda i: (i, 0)),
        out_shape=jax.ShapeDtypeStruct((E, HID), jnp.float32),
    )(e, We, be.reshape(1, HID))


def _edge_mm_fused_body(hg_ref, p_ref, w_ref, b_ref, o_ref):
    t = jnp.maximum(hg_ref[...] + p_ref[...], 0.0)
    o_ref[...] = (
        jnp.dot(t, w_ref[...], preferred_element_type=jnp.float32, precision=lax.Precision.HIGHEST) + b_ref[...]
    )


def edge_mm_fused(h_src, P_prev, We, be):
    E = h_src.shape[0]
    return pl.pallas_call(
        _edge_mm_fused_body,
        grid=(E // BE,),
        in_specs=[
            pl.BlockSpec((BE, HID), lambda i: (i, 0)),
            pl.BlockSpec((BE, HID), lambda i: (i, 0)),
            pl.BlockSpec((HID, HID), lambda i: (0, 0)),
            pl.BlockSpec((1, HID), lambda i: (0, 0)),
        ],
        out_specs=pl.BlockSpec((BE, HID), lambda i: (i, 0)),
        out_shape=jax.ShapeDtypeStruct((E, HID), jnp.float32),
    )(h_src, P_prev, We, be.reshape(1, HID))


def _node_mm_body(h_ref, s_ref, x_ref, wn_ref, bn_ref, wc_ref, bc_ref, o_ref,
                  *, input_relu):
    x = x_ref[...]
    x = jnp.where(jnp.isfinite(x), x, 0.0)
    m = (
        jnp.dot(s_ref[...], wc_ref[0:HID, :], preferred_element_type=jnp.float32, precision=lax.Precision.HIGHEST)
        + jnp.dot(x, wc_ref[HID:2 * HID, :], preferred_element_type=jnp.float32, precision=lax.Precision.HIGHEST)
        + bc_ref[...]
    )
    hin = h_ref[...]
    if input_relu:
        hin = jnp.maximum(hin, 0.0)
    h = (
        jnp.dot(hin, wn_ref[...], preferred_element_type=jnp.float32, precision=lax.Precision.HIGHEST)
        + bn_ref[...]
        + m
    )
    o_ref[...] = h


def node_mm(h, S, X, Wn, bn, Wc, bc, input_relu):
    N, K = h.shape
    return pl.pallas_call(
        functools.partial(_node_mm_body, input_relu=input_relu),
        grid=(N // BN,),
        in_specs=[
            pl.BlockSpec((BN, K), lambda i: (i, 0)),
            pl.BlockSpec((BN, HID), lambda i: (i, 0)),
            pl.BlockSpec((BN, HID), lambda i: (i, 0)),
            pl.BlockSpec((K, HID), lambda i: (0, 0)),
            pl.BlockSpec((1, HID), lambda i: (0, 0)),
            pl.BlockSpec((2 * HID, HID), lambda i: (0, 0)),
            pl.BlockSpec((1, HID), lambda i: (0, 0)),
        ],
        out_specs=pl.BlockSpec((BN, HID), lambda i: (i, 0)),
        out_shape=jax.ShapeDtypeStruct((N, HID), jnp.float32),
    )(h, S, X, Wn, bn.reshape(1, HID), Wc, bc.reshape(1, HID))


def _epilogue_body(h_ref, o_ref, acc_ref, nrm_ref):
    i = pl.program_id(0)

    @pl.when(i == 0)
    def _init():
        acc_ref[...] = jnp.zeros_like(acc_ref)
        nrm_ref[...] = jnp.zeros_like(nrm_ref)

    h = h_ref[...]
    acc_ref[...] += jnp.sum(h, axis=0, keepdims=True)
    rown = jnp.sqrt(jnp.sum(h * h, axis=1, keepdims=True))
    nrm_ref[...] += jnp.sum(rown, axis=0, keepdims=True)

    @pl.when(i == pl.num_programs(0) - 1)
    def _fin():
        factor = jnp.sqrt(jnp.float32(HID)) * N_NODES / nrm_ref[0, 0]
        o_ref[...] = acc_ref[...] * factor


def epilogue(h):
    return pl.pallas_call(
        _epilogue_body,
        grid=(N_NODES // BN,),
        in_specs=[pl.BlockSpec((BN, HID), lambda i: (i, 0))],
        out_specs=pl.BlockSpec((1, HID), lambda i: (0, 0)),
        out_shape=jax.ShapeDtypeStruct((1, HID), jnp.float32),
        scratch_shapes=[
            pltpu.VMEM((1, HID), jnp.float32),
            pltpu.VMEM((1, 1), jnp.float32),
        ],
    )(h)


NR = 320
CH = 128
NPAD = 32 * NR
HREG = HID // 16


def _seg_kernel_body(p_hbm, perm_hbm, offs_hbm, s_hbm, x_hbm,
                     offs_v, perm_v, rows_v, stage_s, stage_x, sem):
    wid = lax.axis_index("s") * 2 + lax.axis_index("c")
    wbase = wid * NR
    nn = jnp.minimum(NR, N_NODES - wbase)

    pltpu.sync_copy(offs_hbm.at[wid], offs_v)

    def sload(i):
        return offs_v[pl.ds(i, 16)][0]

    e0 = sload(0)
    e0a = e0 & ~jnp.int32(7)

    zero = jnp.zeros((16,), jnp.float32)
    ninf = jnp.full((16,), -jnp.inf, jnp.float32)

    def load_chunk(c):
        buf = c & 1
        cs = pl.multiple_of(e0a + c * CH, 8)
        pltpu.sync_copy(perm_hbm.at[pl.ds(cs, CH)], perm_v.at[buf])
        pltpu.async_copy(p_hbm.at[perm_v.at[buf]], rows_v.at[buf], sem).wait()

    load_chunk(jnp.int32(0))

    def node_body(ln, _):
        s_lo = sload(ln)
        s_hi = sload(ln + 1)

        def edge_body(g, accs):
            off = g - e0a
            c = off >> 7
            li = off & (CH - 1)

            @pl.when((li == 0) & (g > e0))
            def _next():
                load_chunk(c)

            buf = c & 1
            out = []
            for k in range(HREG):
                v = rows_v[buf, li, pl.ds(k * 16, 16)]
                out.append(accs[k] + v)
            for k in range(HREG):
                v = rows_v[buf, li, pl.ds(k * 16, 16)]
                out.append(jnp.maximum(accs[HREG + k], v))
            return tuple(out)

        accs = lax.fori_loop(
            s_lo, s_hi, edge_body, tuple([zero] * HREG + [ninf] * HREG))

        sb = (ln >> 3) & 1
        lr = ln & 7
        for k in range(HREG):
            stage_s[sb, lr, pl.ds(k * 16, 16)] = accs[k]
            stage_x[sb, lr, pl.ds(k * 16, 16)] = accs[HREG + k]

        @pl.when((ln & 7) == 7)
        def _flush():
            gb = pl.multiple_of(wbase + ln - 7, 8)
            pltpu.sync_copy(stage_s.at[sb], s_hbm.at[pl.ds(gb, 8)])
            pltpu.sync_copy(stage_x.at[sb], x_hbm.at[pl.ds(gb, 8)])

        return _

    lax.fori_loop(0, nn, node_body, jnp.int32(0))


def seg_reduce(P, perm_pad, offs_tab):
    f = pl.kernel(
        _seg_kernel_body,
        out_type=(
            jax.ShapeDtypeStruct((NPAD, HID), jnp.float32),
            jax.ShapeDtypeStruct((NPAD, HID), jnp.float32),
        ),
        mesh=plsc.VectorSubcoreMesh(core_axis_name="c", subcore_axis_name="s"),
        scratch_types=[
            pltpu.VMEM((344,), jnp.int32),
            pltpu.VMEM((2, CH), jnp.int32),
            pltpu.VMEM((2, CH, HID), jnp.float32),
            pltpu.VMEM((2, 8, HID), jnp.float32),
            pltpu.VMEM((2, 8, HID), jnp.float32),
            pltpu.SemaphoreType.DMA,
        ],
    )
    return f(P, perm_pad, offs_tab)


def kernel(x, edge_index, edge_attr,
           Wn0, bn0, We0, be0, Wc0, bc0,
           Wn1, bn1, We1, be1, Wc1, bc1,
           Wn2, bn2, We2, be2, Wc2, bc2):
    src = edge_index[0]
    dst = edge_index[1]

    perm = jnp.argsort(dst).astype(jnp.int32)
    dst_sorted = dst[perm]
    offs_g = jnp.searchsorted(
        dst_sorted, jnp.arange(NPAD + 1, dtype=jnp.int32), side="left"
    ).astype(jnp.int32)
    tab_idx = jnp.clip(
        jnp.arange(32, dtype=jnp.int32)[:, None] * NR
        + jnp.arange(344, dtype=jnp.int32)[None, :], 0, NPAD)
    offs_tab = offs_g[tab_idx]
    perm_pad = jnp.pad(perm, (0, CH))

    P = edge_mm(edge_attr, We0, be0)
    params = [(Wn0, bn0, We1, be1, Wc0, bc0),
              (Wn1, bn1, We2, be2, Wc1, bc1),
              (Wn2, bn2, None, None, Wc2, bc2)]
    h = x
    for i, (Wn, bn, We_next, be_next, Wc, bc) in enumerate(params):
        S, X = seg_reduce(P, perm_pad, offs_tab)
        S, X = S[:N_NODES], X[:N_NODES]
        h = node_mm(h, S, X, Wn, bn, Wc, bc, input_relu=(i > 0))
        if We_next is not None:
            h_src = h[src]
            P = edge_mm_fused(h_src, P, We_next, be_next)
    return epilogue(h)
